# baseline (speedup 1.0000x reference)
import functools

import jax
import jax.numpy as jnp
from jax import lax
from jax.experimental import pallas as pl
from jax.experimental.pallas import tpu as pltpu
from jax.experimental.pallas import tpu_sc as plsc

_NSUB = 16
_L = 16


def _sc_stage(props, gt_boxes, rb2, rc2, labels, n_rows, n_gt):
    per = -(-n_rows // (_NSUB * _L)) * _L
    chunks = per // _L
    n_full = n_rows // per
    tail_chunks = (n_rows - n_full * per) // _L
    n_cls = rc2.shape[1]
    ngc = (n_gt + _L - 1) // _L

    def body(props_hbm, gt_hbm, rb_hbm, rc_hbm, lab_hbm,
             rvb_out, rvc_out, rvp_out, misc_out,
             prop_v, gt_v, lab_v, pmax_v, pidx_v, shared_max, shared_idx,
             allmax_v, allidx_v, rvb_v, rvc_v, rvp_v, misc_v, sem):
        sid = lax.axis_index("s")
        base = sid * per
        my_chunks = jnp.where(sid == n_full, tail_chunks, chunks)
        pltpu.sync_copy(props_hbm.at[pl.ds(base, per)], prop_v)
        pltpu.sync_copy(gt_hbm, gt_v)
        iota = lax.iota(jnp.int32, _L)
        col = [jnp.full((_L,), c, jnp.int32) for c in range(4)]

        graw = []
        gcor = []
        for cc in range(ngc):
            rows = cc * _L + iota
            msk = rows < n_gt
            cy = plsc.load_gather(gt_v, [rows, col[0]], mask=msk)
            cx = plsc.load_gather(gt_v, [rows, col[1]], mask=msk)
            hh = plsc.load_gather(gt_v, [rows, col[2]], mask=msk)
            ww = plsc.load_gather(gt_v, [rows, col[3]], mask=msk)
            gy1 = cy - hh * 0.5
            gx1 = cx - ww * 0.5
            gy2 = cy + hh * 0.5
            gx2 = cx + ww * 0.5
            gab = (gy2 - gy1) * (gx2 - gx1)
            graw.append((cy, cx, hh, ww))
            gcor.append((gy1, gx1, gy2, gx2, gab))

        pm = [jnp.zeros((_L,), jnp.float32) for _ in range(ngc)]
        pi = [jnp.zeros((_L,), jnp.int32) for _ in range(ngc)]
        neg_inf = jnp.float32(-jnp.inf)
        for j in range(n_gt):
            cc, l = divmod(j, _L)
            lane = iota == l

            def bc(v):
                return jnp.full((_L,), jnp.max(jnp.where(lane, v, neg_inf)),
                                jnp.float32)

            gy1b, gx1b, gy2b, gx2b, gabb = [bc(v) for v in gcor[cc]]

            def loop(c, carry):
                m, mi = carry
                rows = c * _L + iota
                py1 = plsc.load_gather(prop_v, [rows, col[0]])
                px1 = plsc.load_gather(prop_v, [rows, col[1]])
                py2 = plsc.load_gather(prop_v, [rows, col[2]])
                px2 = plsc.load_gather(prop_v, [rows, col[3]])
                aa = (py2 - py1) * (px2 - px1)
                iy1 = jnp.maximum(py1, gy1b)
                ix1 = jnp.maximum(px1, gx1b)
                iy2 = jnp.minimum(py2, gy2b)
                ix2 = jnp.minimum(px2, gx2b)
                ih = jnp.maximum(iy2 - iy1, 0.0)
                iw = jnp.maximum(ix2 - ix1, 0.0)
                inter = ih * iw
                iou = inter / (aa + gabb - inter)
                upd = iou > m
                gidx = base + rows
                return jnp.where(upd, iou, m), jnp.where(upd, gidx, mi)

            m, mi = lax.fori_loop(
                0, my_chunks, loop,
                (jnp.full((_L,), -1.0, jnp.float32),
                 jnp.zeros((_L,), jnp.int32)))
            mm = jnp.max(m)
            cand = jnp.where(m == mm, mi, jnp.int32(2**30))
            bi = jnp.min(cand)
            pm[cc] = jnp.where(lane, mm, pm[cc])
            pi[cc] = jnp.where(lane, bi, pi[cc])

        for cc in range(ngc):
            pmax_v[cc, :] = pm[cc]
            pidx_v[cc, :] = pi[cc]
        pltpu.sync_copy(pmax_v, shared_max.at[sid])
        pltpu.sync_copy(pidx_v, shared_idx.at[sid])
        plsc.subcore_barrier()

        @pl.when(sid == 0)
        def _():
            pltpu.sync_copy(shared_max, allmax_v)
            pltpu.sync_copy(shared_idx, allidx_v)
            pltpu.sync_copy(lab_hbm, lab_v.at[pl.ds(0, n_gt)])
            cis = []
            for cc in range(ngc):
                cm = jnp.full((_L,), -2.0, jnp.float32)
                ci = jnp.zeros((_L,), jnp.int32)
                for w in range(_NSUB):
                    vm = allmax_v[w, cc, :]
                    vi = allidx_v[w, cc, :]
                    better = (vm > cm) | ((vm == cm) & (vi < ci))
                    cm = jnp.where(better, vm, cm)
                    ci = jnp.where(better, vi, ci)
                cis.append(ci)
                misc_v[0, pl.ds(cc * _L, _L)] = ci.astype(jnp.float32)
                lab = lab_v[pl.ds(cc * _L, _L)]
                misc_v[1, pl.ds(cc * _L, _L)] = lab.astype(jnp.float32)
                for r in range(4):
                    misc_v[2 + r, pl.ds(cc * _L, _L)] = graw[cc][r]
            cps = []
            for j in range(n_gt):
                cc, l = divmod(j, _L)
                sel = iota == l
                idx_j = jnp.max(jnp.where(sel, cis[cc], 0))
                cps.append(pltpu.async_copy(
                    rb_hbm.at[pl.ds(idx_j, 1)], rvb_v.at[pl.ds(j, 1)], sem))
                cps.append(pltpu.async_copy(
                    rc_hbm.at[pl.ds(idx_j, 1)], rvc_v.at[pl.ds(j, 1)], sem))
                cps.append(pltpu.async_copy(
                    props_hbm.at[pl.ds(idx_j, 1)], rvp_v.at[pl.ds(j, 1)], sem))
            for cp in cps:
                cp.wait()
            pltpu.sync_copy(rvb_v, rvb_out)
            pltpu.sync_copy(rvc_v, rvc_out)
            pltpu.sync_copy(rvp_v, rvp_out)
            pltpu.sync_copy(misc_v, misc_out)

    mesh = plsc.VectorSubcoreMesh(core_axis_name="c", subcore_axis_name="s",
                                  num_cores=1, num_subcores=_NSUB)
    f = pl.kernel(
        body, mesh=mesh,
        compiler_params=pltpu.CompilerParams(needs_layout_passes=False,
                                             use_tc_tiling_on_sc=False),
        out_type=(jax.ShapeDtypeStruct((2 * _L, 4), jnp.float32),
                  jax.ShapeDtypeStruct((2 * _L, n_cls), jnp.float32),
                  jax.ShapeDtypeStruct((2 * _L, 4), jnp.float32),
                  jax.ShapeDtypeStruct((6, 2 * _L), jnp.float32)),
        scratch_types=[
            pltpu.VMEM((per, 4), jnp.float32),
            pltpu.VMEM((n_gt, 4), jnp.float32),
            pltpu.VMEM((2 * _L,), jnp.int32),
            pltpu.VMEM((ngc, _L), jnp.float32),
            pltpu.VMEM((ngc, _L), jnp.int32),
            pltpu.VMEM_SHARED((_NSUB, ngc, _L), jnp.float32),
            pltpu.VMEM_SHARED((_NSUB, ngc, _L), jnp.int32),
            pltpu.VMEM((_NSUB, ngc, _L), jnp.float32),
            pltpu.VMEM((_NSUB, ngc, _L), jnp.int32),
            pltpu.VMEM((2 * _L, 4), jnp.float32),
            pltpu.VMEM((2 * _L, n_cls), jnp.float32),
            pltpu.VMEM((2 * _L, 4), jnp.float32),
            pltpu.VMEM((6, 2 * _L), jnp.float32),
            pltpu.SemaphoreType.DMA,
        ])
    return f(props, gt_boxes, rb2, rc2, labels)


def _tc_body(rvb_ref, rvc_ref, rvp_ref, misc_ref, out_ref, *, n_gt, n_cls):
    M, C = n_gt, n_cls
    rvb = rvb_ref[...][:M, :]
    rvc = rvc_ref[...][:M, :]
    rvp = rvp_ref[...][:M, :]
    misc = misc_ref[...]
    gi = misc[0:1, :M]
    lab = misc[1:2, :M]
    gcy = misc[2:3, :M]
    gcx = misc[3:4, :M]
    gh = misc[4:5, :M]
    gw = misc[5:6, :M]

    p_y1 = rvp[:, 0:1]
    p_x1 = rvp[:, 1:2]
    p_y2 = rvp[:, 2:3]
    p_x2 = rvp[:, 3:4]
    r_cy = (p_y1 + p_y2) * 0.5
    r_cx = (p_x1 + p_x2) * 0.5
    r_h = p_y2 - p_y1
    r_w = p_x2 - p_x1
    gy1 = gcy - gh * 0.5
    gx1 = gcx - gw * 0.5
    gy2 = gcy + gh * 0.5
    gx2 = gcx + gw * 0.5
    area_b = (gy2 - gy1) * (gx2 - gx1)
    area_a = (p_y2 - p_y1) * (p_x2 - p_x1)
    iy1 = jnp.maximum(p_y1, gy1)
    ix1 = jnp.maximum(p_x1, gx1)
    iy2 = jnp.minimum(p_y2, gy2)
    ix2 = jnp.minimum(p_x2, gx2)
    ih = jnp.maximum(iy2 - iy1, 0.0)
    iw = jnp.maximum(ix2 - ix1, 0.0)
    inter = ih * iw
    iou = inter / (area_a + area_b - inter)

    iotaC = lax.broadcasted_iota(jnp.int32, (M, M), 1)
    iotaR = lax.broadcasted_iota(jnp.int32, (M, M), 0)
    mrow = jnp.max(iou, axis=1, keepdims=True)
    g = jnp.min(jnp.where(iou == mrow, iotaC, jnp.int32(2**30)),
                axis=1, keepdims=True)
    onehot = (iotaC == g).astype(jnp.float32)
    tcy = jnp.sum(onehot * gcy, axis=1, keepdims=True)
    tcx = jnp.sum(onehot * gcx, axis=1, keepdims=True)
    th = jnp.sum(onehot * gh, axis=1, keepdims=True)
    tw = jnp.sum(onehot * gw, axis=1, keepdims=True)
    ty = (tcy - r_cy) / r_h
    tx = (tcx - r_cx) / r_w
    tlh = jnp.log(th / r_h)
    tlw = jnp.log(tw / r_w)
    labsel = jnp.sum(onehot * lab, axis=1, keepdims=True)

    ident = (iotaR == iotaC).astype(jnp.float32)
    A = jnp.broadcast_to(gi, (M, M))
    gcol = jnp.sum(A * ident, axis=1, keepdims=True)
    dup = (A == gcol) & (iotaC < iotaR)
    isf = jnp.sum(dup.astype(jnp.float32), axis=1, keepdims=True) == 0.0
    cnt = jnp.sum(isf.astype(jnp.float32))

    rb_y = rvb[:, 0:1]
    rb_x = rvb[:, 1:2]
    rb_h = rvb[:, 2:3]
    rb_w = rvb[:, 3:4]
    lx = jnp.sum(jnp.where(isf, jnp.abs(rb_x - tx), 0.0)) / cnt
    ly = jnp.sum(jnp.where(isf, jnp.abs(rb_y - ty), 0.0)) / cnt
    lh = jnp.sum(jnp.where(isf, jnp.abs(rb_h - tlh), 0.0)) / cnt
    lw = jnp.sum(jnp.where(isf, jnp.abs(rb_w - tlw), 0.0)) / cnt

    rmax = jnp.max(rvc, axis=1, keepdims=True)
    lse = jnp.log(jnp.sum(jnp.exp(rvc - rmax), axis=1, keepdims=True)) + rmax
    iotaCls = lax.broadcasted_iota(jnp.int32, (M, C), 1).astype(jnp.float32)
    picked = jnp.sum(jnp.where(iotaCls == labsel, rvc, 0.0),
                     axis=1, keepdims=True)
    lc = jnp.sum(jnp.where(isf, lse - picked, 0.0)) / cnt

    il = lax.broadcasted_iota(jnp.int32, (1, 128), 1)
    acc = jnp.where(il == 0, lx, 0.0)
    acc = jnp.where(il == 1, ly, acc)
    acc = jnp.where(il == 2, lh, acc)
    acc = jnp.where(il == 3, lw, acc)
    acc = jnp.where(il == 4, lc, acc)
    out_ref[...] = acc


def _tc_stage(rvb, rvc, rvp, misc, n_gt, n_cls, interpret=False):
    return pl.pallas_call(
        functools.partial(_tc_body, n_gt=n_gt, n_cls=n_cls),
        out_shape=jax.ShapeDtypeStruct((1, 128), jnp.float32),
        interpret=interpret,
    )(rvb, rvc, rvp, misc)


def kernel(proposal_rois, rcnn_boxes, rcnn_class, gt_boxes, gt_labels):
    N = proposal_rois.shape[0]
    M = gt_boxes.shape[0]
    C = rcnn_class.shape[-1]
    rb2 = rcnn_boxes.reshape(N, 4)
    rc2 = rcnn_class.reshape(N, C)
    rvb, rvc, rvp, misc = _sc_stage(proposal_rois, gt_boxes, rb2, rc2,
                                    gt_labels, N, M)
    out = _tc_stage(rvb, rvc, rvp, misc, M, C)
    return tuple(out[0, k].reshape(()) for k in range(5))


# speedup vs baseline: 2.4650x; 2.4650x over previous
import functools

import jax
import jax.numpy as jnp
from jax import lax
from jax.experimental import pallas as pl
from jax.experimental.pallas import tpu as pltpu
from jax.experimental.pallas import tpu_sc as plsc

_NSUB = 16
_L = 16
_W = 32


def _sc_stage(pf2, gt_boxes, rbf2, rcf2, labels, n_rows, n_gt):
    per = -(-n_rows // (_NSUB * _L)) * _L
    rows40 = per // _W
    plane = n_rows // _W
    n_cls = rcf2.shape[0] * _W // n_rows
    ngc = (n_gt + _L - 1) // _L
    n_ck = (n_cls + 3) // 4

    def body(pf_hbm, gt_hbm, rb_hbm, rc_hbm, lab_hbm,
             bp_out, rc_out, misc_out,
             prop_v, gt_v, lab_v, pmax_v, pidx_v, shared_max, shared_idx,
             allmax_v, allidx_v, pidx_ref, bidx_ref, cidx_ref,
             pwin, bwin, cwin, bp_v, rcg_v, misc_v, sem):
        sid = lax.axis_index("s")
        base = jnp.where(sid == _NSUB - 1, n_rows - per, sid * per)
        rbase = base // _W
        for c in range(4):
            pltpu.sync_copy(pf_hbm.at[pl.ds(c * plane + rbase, rows40)],
                            prop_v.at[c])
        pltpu.sync_copy(gt_hbm, gt_v)
        iota = lax.iota(jnp.int32, _L)
        col = [jnp.full((_L,), c, jnp.int32) for c in range(4)]

        gcor = []
        for cc in range(ngc):
            rows = cc * _L + iota
            msk = rows < n_gt
            cy = plsc.load_gather(gt_v, [rows, col[0]], mask=msk)
            cx = plsc.load_gather(gt_v, [rows, col[1]], mask=msk)
            hh = plsc.load_gather(gt_v, [rows, col[2]], mask=msk)
            ww = plsc.load_gather(gt_v, [rows, col[3]], mask=msk)
            gy1 = cy - hh * 0.5
            gx1 = cx - ww * 0.5
            gy2 = cy + hh * 0.5
            gx2 = cx + ww * 0.5
            gab = (gy2 - gy1) * (gx2 - gx1)
            gcor.append((gy1, gx1, gy2, gx2, gab))

        pm = [jnp.zeros((_L,), jnp.float32) for _ in range(ngc)]
        pi = [jnp.zeros((_L,), jnp.int32) for _ in range(ngc)]
        neg_inf = jnp.float32(-jnp.inf)
        for j in range(n_gt):
            cc, l = divmod(j, _L)
            lane = iota == l

            def bc(v):
                return jnp.full((_L,), jnp.max(jnp.where(lane, v, neg_inf)),
                                jnp.float32)

            gy1b, gx1b, gy2b, gx2b, gabb = [bc(v) for v in gcor[cc]]

            def loop(r, carry):
                m, mi = carry
                for half in range(2):
                    off = half * _L
                    py1 = prop_v[0, r, pl.ds(off, _L)]
                    px1 = prop_v[1, r, pl.ds(off, _L)]
                    py2 = prop_v[2, r, pl.ds(off, _L)]
                    px2 = prop_v[3, r, pl.ds(off, _L)]
                    aa = (py2 - py1) * (px2 - px1)
                    iy1 = jnp.maximum(py1, gy1b)
                    ix1 = jnp.maximum(px1, gx1b)
                    iy2 = jnp.minimum(py2, gy2b)
                    ix2 = jnp.minimum(px2, gx2b)
                    ih = jnp.maximum(iy2 - iy1, 0.0)
                    iw = jnp.maximum(ix2 - ix1, 0.0)
                    inter = ih * iw
                    iou = inter / (aa + gabb - inter)
                    upd = iou > m
                    gidx = (base + off + iota) + r * _W
                    m = jnp.where(upd, iou, m)
                    mi = jnp.where(upd, gidx, mi)
                return m, mi

            m, mi = lax.fori_loop(
                0, rows40, loop,
                (jnp.full((_L,), -1.0, jnp.float32),
                 jnp.zeros((_L,), jnp.int32)))
            mm = jnp.max(m)
            cand = jnp.where(m == mm, mi, jnp.int32(2**30))
            bi = jnp.min(cand)
            pm[cc] = jnp.where(lane, mm, pm[cc])
            pi[cc] = jnp.where(lane, bi, pi[cc])

        for cc in range(ngc):
            pmax_v[cc, :] = pm[cc]
            pidx_v[cc, :] = pi[cc]
        pltpu.sync_copy(pmax_v, shared_max.at[sid])
        pltpu.sync_copy(pidx_v, shared_idx.at[sid])
        plsc.subcore_barrier()

        @pl.when(sid == 0)
        def _():
            pltpu.sync_copy(shared_max, allmax_v)
            pltpu.sync_copy(shared_idx, allidx_v)
            pltpu.sync_copy(lab_hbm, lab_v.at[pl.ds(0, n_gt)])
            cis = []
            for cc in range(ngc):
                cm = jnp.full((_L,), -2.0, jnp.float32)
                ci = jnp.zeros((_L,), jnp.int32)
                for w in range(_NSUB):
                    vm = allmax_v[w, cc, :]
                    vi = allidx_v[w, cc, :]
                    better = (vm > cm) | ((vm == cm) & (vi < ci))
                    cm = jnp.where(better, vm, cm)
                    ci = jnp.where(better, vi, ci)
                cis.append(ci)
                misc_v[0, pl.ds(cc * _L, _L)] = ci.astype(jnp.float32)
                lab = lab_v[pl.ds(cc * _L, _L)]
                misc_v[1, pl.ds(cc * _L, _L)] = lab.astype(jnp.float32)
            colv = [ci & (_W - 1) for ci in cis]
            rowv = [lax.shift_right_logical(ci, 5) for ci in cis]
            for c in range(4):
                for cc in range(ngc):
                    ent = c * plane + rowv[cc]
                    pidx_ref[pl.ds((c * ngc + cc) * _L, _L)] = ent
                    bidx_ref[pl.ds((c * ngc + cc) * _L, _L)] = ent
            for c in range(4 * n_ck):
                for cc in range(ngc):
                    ent = (c * plane + rowv[cc] if c < n_cls
                           else jnp.zeros((_L,), jnp.int32))
                    cidx_ref[pl.ds((c * ngc + cc) * _L, _L)] = ent
            cps = [pltpu.async_copy(pf_hbm.at[pidx_ref], pwin, sem),
                   pltpu.async_copy(rb_hbm.at[bidx_ref], bwin, sem)]
            for k in range(n_ck):
                cps.append(pltpu.async_copy(
                    rc_hbm.at[cidx_ref.at[pl.ds(k * 128, 128)]],
                    cwin.at[pl.ds(k * 128, 128)], sem))
            for cp in cps:
                cp.wait()
            for cc in range(ngc):
                for c in range(4):
                    rsel = (c * ngc + cc) * _L + iota
                    bp_v[c, pl.ds(cc * _L, _L)] = plsc.load_gather(
                        bwin, [rsel, colv[cc]])
                    bp_v[4 + c, pl.ds(cc * _L, _L)] = plsc.load_gather(
                        pwin, [rsel, colv[cc]])
                for c in range(n_cls):
                    rsel = (c * ngc + cc) * _L + iota
                    rcg_v[c, pl.ds(cc * _L, _L)] = plsc.load_gather(
                        cwin, [rsel, colv[cc]])
            pltpu.sync_copy(bp_v, bp_out)
            pltpu.sync_copy(rcg_v, rc_out)
            pltpu.sync_copy(misc_v, misc_out)

    mesh = plsc.VectorSubcoreMesh(core_axis_name="c", subcore_axis_name="s",
                                  num_cores=1, num_subcores=_NSUB)
    f = pl.kernel(
        body, mesh=mesh,
        compiler_params=pltpu.CompilerParams(needs_layout_passes=False,
                                             use_tc_tiling_on_sc=False),
        out_type=(jax.ShapeDtypeStruct((8, 2 * _L), jnp.float32),
                  jax.ShapeDtypeStruct((24, 2 * _L), jnp.float32),
                  jax.ShapeDtypeStruct((8, 2 * _L), jnp.float32)),
        scratch_types=[
            pltpu.VMEM((4, rows40, _W), jnp.float32),
            pltpu.VMEM((n_gt, 4), jnp.float32),
            pltpu.VMEM((2 * _L,), jnp.int32),
            pltpu.VMEM((ngc, _L), jnp.float32),
            pltpu.VMEM((ngc, _L), jnp.int32),
            pltpu.VMEM_SHARED((_NSUB, ngc, _L), jnp.float32),
            pltpu.VMEM_SHARED((_NSUB, ngc, _L), jnp.int32),
            pltpu.VMEM((_NSUB, ngc, _L), jnp.float32),
            pltpu.VMEM((_NSUB, ngc, _L), jnp.int32),
            pltpu.VMEM((4 * ngc * _L,), jnp.int32),
            pltpu.VMEM((4 * ngc * _L,), jnp.int32),
            pltpu.VMEM((4 * n_ck * ngc * _L,), jnp.int32),
            pltpu.VMEM((4 * ngc * _L, _W), jnp.float32),
            pltpu.VMEM((4 * ngc * _L, _W), jnp.float32),
            pltpu.VMEM((4 * n_ck * ngc * _L, _W), jnp.float32),
            pltpu.VMEM((8, 2 * _L), jnp.float32),
            pltpu.VMEM((24, 2 * _L), jnp.float32),
            pltpu.VMEM((8, 2 * _L), jnp.float32),
            pltpu.SemaphoreType.DMA,
        ])
    return f(pf2, gt_boxes, rbf2, rcf2, labels)


def _tc_body(bp_ref, rcg_ref, misc_ref, gt_ref, out_ref, *, n_gt, n_cls):
    M, C = n_gt, n_cls
    bp = bp_ref[...]
    rc_t = rcg_ref[...][:C, :M]
    misc = misc_ref[...]
    gt = gt_ref[...]
    gi = misc[0:1, :M]
    labr = misc[1:2, :M]

    rb_y = bp[0:1, :M]
    rb_x = bp[1:2, :M]
    rb_h = bp[2:3, :M]
    rb_w = bp[3:4, :M]
    p_y1 = bp[4:5, :M]
    p_x1 = bp[5:6, :M]
    p_y2 = bp[6:7, :M]
    p_x2 = bp[7:8, :M]
    r_cy = (p_y1 + p_y2) * 0.5
    r_cx = (p_x1 + p_x2) * 0.5
    r_h = p_y2 - p_y1
    r_w = p_x2 - p_x1
    gcy = gt[:, 0:1]
    gcx = gt[:, 1:2]
    gh = gt[:, 2:3]
    gw = gt[:, 3:4]
    gy1 = gcy - gh * 0.5
    gx1 = gcx - gw * 0.5
    gy2 = gcy + gh * 0.5
    gx2 = gcx + gw * 0.5
    area_b = (gy2 - gy1) * (gx2 - gx1)
    area_a = (p_y2 - p_y1) * (p_x2 - p_x1)
    iy1 = jnp.maximum(p_y1, gy1)
    ix1 = jnp.maximum(p_x1, gx1)
    iy2 = jnp.minimum(p_y2, gy2)
    ix2 = jnp.minimum(p_x2, gx2)
    ih = jnp.maximum(iy2 - iy1, 0.0)
    iw = jnp.maximum(ix2 - ix1, 0.0)
    inter = ih * iw
    iou = inter / (area_a + area_b - inter)

    iotaC = lax.broadcasted_iota(jnp.int32, (M, M), 1)
    iotaR = lax.broadcasted_iota(jnp.int32, (M, M), 0)
    mcol = jnp.max(iou, axis=0, keepdims=True)
    g = jnp.min(jnp.where(iou == mcol, iotaR, jnp.int32(2**30)),
                axis=0, keepdims=True)
    onehot = (iotaR == g).astype(jnp.float32)
    tcy = jnp.sum(onehot * gcy, axis=0, keepdims=True)
    tcx = jnp.sum(onehot * gcx, axis=0, keepdims=True)
    th = jnp.sum(onehot * gh, axis=0, keepdims=True)
    tw = jnp.sum(onehot * gw, axis=0, keepdims=True)
    ty = (tcy - r_cy) / r_h
    tx = (tcx - r_cx) / r_w
    tlh = jnp.log(th / r_h)
    tlw = jnp.log(tw / r_w)
    ident = (iotaR == iotaC).astype(jnp.float32)
    labcol = jnp.sum(jnp.broadcast_to(labr, (M, M)) * ident,
                     axis=1, keepdims=True)
    labsel = jnp.sum(onehot * labcol, axis=0, keepdims=True)

    A = jnp.broadcast_to(gi, (M, M))
    gi_col = jnp.sum(A * ident, axis=1, keepdims=True)
    dup = (A == gi_col) & (iotaR < iotaC)
    isf = jnp.sum(dup.astype(jnp.float32), axis=0, keepdims=True) == 0.0
    cnt = jnp.sum(isf.astype(jnp.float32))

    lx = jnp.sum(jnp.where(isf, jnp.abs(rb_x - tx), 0.0)) / cnt
    ly = jnp.sum(jnp.where(isf, jnp.abs(rb_y - ty), 0.0)) / cnt
    lh = jnp.sum(jnp.where(isf, jnp.abs(rb_h - tlh), 0.0)) / cnt
    lw = jnp.sum(jnp.where(isf, jnp.abs(rb_w - tlw), 0.0)) / cnt

    rmax = jnp.max(rc_t, axis=0, keepdims=True)
    lse = jnp.log(jnp.sum(jnp.exp(rc_t - rmax), axis=0, keepdims=True)) + rmax
    iotaCls = lax.broadcasted_iota(jnp.int32, (C, M), 0).astype(jnp.float32)
    picked = jnp.sum(jnp.where(iotaCls == labsel, rc_t, 0.0),
                     axis=0, keepdims=True)
    lc = jnp.sum(jnp.where(isf, lse - picked, 0.0)) / cnt

    il = lax.broadcasted_iota(jnp.int32, (1, 128), 1)
    acc = jnp.where(il == 0, lx, 0.0)
    acc = jnp.where(il == 1, ly, acc)
    acc = jnp.where(il == 2, lh, acc)
    acc = jnp.where(il == 3, lw, acc)
    acc = jnp.where(il == 4, lc, acc)
    out_ref[...] = acc


def _tc_stage(bp, rcg, misc, gt, n_gt, n_cls, interpret=False):
    return pl.pallas_call(
        functools.partial(_tc_body, n_gt=n_gt, n_cls=n_cls),
        out_shape=jax.ShapeDtypeStruct((1, 128), jnp.float32),
        interpret=interpret,
    )(bp, rcg, misc, gt)


def kernel(proposal_rois, rcnn_boxes, rcnn_class, gt_boxes, gt_labels):
    N = proposal_rois.shape[0]
    M = gt_boxes.shape[0]
    C = rcnn_class.shape[-1]
    pf2 = proposal_rois.T.reshape(4 * N // _W, _W)
    rbf2 = rcnn_boxes.reshape(N, 4).T.reshape(4 * N // _W, _W)
    rcf2 = rcnn_class.reshape(N, C).T.reshape(C * N // _W, _W)
    bp, rcg, misc = _sc_stage(pf2, gt_boxes, rbf2, rcf2, gt_labels, N, M)
    out = _tc_stage(bp, rcg, misc, gt_boxes, M, C)
    return tuple(out[0, k].reshape(()) for k in range(5))
